# baseline (speedup 1.0000x reference)
import functools

import jax
import jax.numpy as jnp
from jax import lax
from jax.experimental import pallas as pl
from jax.experimental.pallas import tpu as pltpu
from jax.experimental.pallas import tpu_sc as plsc

_N = 10000
_E = 320000
_D = 128
_A = 33

_NC = 2
_NS = 16
_NW = _NC * _NS
_C = 640
_K = _C // 128
_CPW = 16
_EPW = _C * _CPW
_E_PAD = _NW * _EPW
_HALF = 5120
_ACC_R = 5248
_T = 512


def _sc_gather(xn, idx_src2, idx_dst2):
    mesh = plsc.VectorSubcoreMesh(core_axis_name="c", subcore_axis_name="s")

    @functools.partial(
        pl.kernel,
        out_type=[jax.ShapeDtypeStruct((_E_PAD, _D), jnp.float32),
                  jax.ShapeDtypeStruct((_E_PAD, _D), jnp.float32)],
        mesh=mesh,
        scratch_types=[pltpu.VMEM((8, 128), jnp.int32),
                       pltpu.VMEM((_C, _D), jnp.float32),
                       pltpu.SemaphoreType.DMA],
    )
    def k(xn_hbm, is_hbm, id_hbm, os_hbm, od_hbm, idx_v, rows_v, sem):
        w = lax.axis_index("s") * _NC + lax.axis_index("c")
        base = w * _EPW

        def run(i_hbm, o_hbm):
            def body(ck, carry):
                off = base + ck * _C
                pltpu.sync_copy(i_hbm.at[w * _CPW + ck], idx_v)
                descs = [
                    pltpu.async_copy(xn_hbm.at[idx_v.at[j]],
                                     rows_v.at[pl.ds(j * 128, 128)], sem)
                    for j in range(_K)
                ]
                for d in descs:
                    d.wait()
                pltpu.sync_copy(rows_v, o_hbm.at[pl.ds(off, _C)])
                return carry

            lax.fori_loop(0, _CPW, body, 0)

        run(is_hbm, os_hbm)
        run(id_hbm, od_hbm)

    return k(xn, idx_src2, idx_dst2)


def _sc_scatter(a_dst, a_src, idx_dst2, idx_src2, zrows):
    mesh = plsc.VectorSubcoreMesh(core_axis_name="c", subcore_axis_name="s")
    z_per_sub = _ACC_R // _NS
    o_per_sub = _HALF // _NS
    cpw = _E_PAD // (_NS * _C)
    epw = cpw * _C

    @functools.partial(
        pl.kernel,
        out_type=jax.ShapeDtypeStruct((2 * _HALF, _D), jnp.float32),
        mesh=mesh,
        scratch_types=[pltpu.VMEM((8, 128), jnp.int32),
                       pltpu.VMEM((_C, _D), jnp.float32),
                       pltpu.VMEM_SHARED((_ACC_R, _D), jnp.float32)],
    )
    def k(ad_hbm, as_hbm, id_hbm, is_hbm, z_hbm, out_hbm, idx_v, rows_v, acc):
        c = lax.axis_index("c")
        s = lax.axis_index("s")
        base = s * epw

        pltpu.sync_copy(z_hbm.at[pl.ds(s * z_per_sub, z_per_sub)],
                        acc.at[pl.ds(s * z_per_sub, z_per_sub)])
        plsc.subcore_barrier()

        def run(r_hbm, i_hbm):
            def body(ck, carry):
                off = base + ck * _C
                pltpu.sync_copy(i_hbm.at[(c * _NS + s) * cpw + ck], idx_v)
                pltpu.sync_copy(r_hbm.at[pl.ds(off, _C)], rows_v)
                for j in range(_K):
                    pltpu.sync_copy(rows_v.at[pl.ds(j * 128, 128)],
                                    acc.at[idx_v.at[j]], add=True)
                return carry

            lax.fori_loop(0, cpw, body, 0)

        run(ad_hbm, id_hbm)
        run(as_hbm, is_hbm)
        plsc.subcore_barrier()
        pltpu.sync_copy(acc.at[pl.ds(s * o_per_sub, o_per_sub)],
                        out_hbm.at[pl.ds(c * _HALF + s * o_per_sub,
                                         o_per_sub)])

    return k(a_dst, a_src, idx_dst2, idx_src2, zrows)


def _tc_edge_body(attr_ref, xs_ref, xd_ref, w1_ref, b1_ref, w2_ref,
                  adst_ref, asrc_ref):
    attr = attr_ref[...]
    W = jnp.dot(attr, w1_ref[...], preferred_element_type=jnp.float32)
    W = W + b1_ref[...]
    W = W * jax.nn.sigmoid(W)
    xs = xs_ref[...]
    xd = xd_ref[...]
    g = W * (xs - xd)
    a = W * (xs + xd) * 0.5
    dxe = jnp.concatenate([g, a, g * a, g * g, a * a], axis=1)
    w2 = w2_ref[...]
    x = jnp.tanh(dxe).astype(jnp.bfloat16)
    x = jnp.dot(x, w2, preferred_element_type=jnp.float32)
    x = x - jnp.mean(x, axis=1, keepdims=True)
    x = x * lax.rsqrt(jnp.sum(x * x, axis=1, keepdims=True) + 0.001)
    x = jnp.tanh(x).astype(jnp.bfloat16)
    x = jnp.dot(x, w2, preferred_element_type=jnp.float32)
    x = jnp.tanh(x)
    x0 = x[:, :_D]
    s = 0.5 * (x[:, _D:2 * _D] + x[:, 2 * _D:3 * _D]
               + x[:, 3 * _D:4 * _D] + x[:, 4 * _D:])
    adst_ref[...] = W * (s + x0)
    asrc_ref[...] = W * (s - x0)


def _tc_edges(xe_attr, xs, xd, fc1_wT, fc1_b2, dl_w1T):
    grid = _E_PAD // _T
    last_real = _E // _T - 1
    return pl.pallas_call(
        _tc_edge_body,
        grid=(grid,),
        in_specs=[
            pl.BlockSpec((_T, _A), lambda i: (jnp.minimum(i, last_real), 0)),
            pl.BlockSpec((_T, _D), lambda i: (i, 0)),
            pl.BlockSpec((_T, _D), lambda i: (i, 0)),
            pl.BlockSpec((_A, _D), lambda i: (0, 0)),
            pl.BlockSpec((1, _D), lambda i: (0, 0)),
            pl.BlockSpec((5 * _D, 5 * _D), lambda i: (0, 0)),
        ],
        out_specs=[
            pl.BlockSpec((_T, _D), lambda i: (i, 0)),
            pl.BlockSpec((_T, _D), lambda i: (i, 0)),
        ],
        out_shape=[jax.ShapeDtypeStruct((_E_PAD, _D), jnp.float32),
                   jax.ShapeDtypeStruct((_E_PAD, _D), jnp.float32)],
        compiler_params=pltpu.CompilerParams(
            dimension_semantics=("arbitrary",)),
    )(xe_attr, xs, xd, fc1_wT, fc1_b2, dl_w1T)


def _chunk_idx(idx_pad):
    a = idx_pad.reshape(-1, _K, 128)
    pad = [(0, 0)] * (a.ndim - 2) + [(0, 8 - _K), (0, 0)]
    return jnp.pad(a, pad)


def _local_idx(idx_pad):
    halves = []
    for core in range(_NC):
        loc = idx_pad - core * _HALF
        ok = (loc >= 0) & (loc < _HALF)
        halves.append(jnp.where(ok, loc, _HALF))
    return _chunk_idx(jnp.stack(halves).reshape(-1, _K, 128))


def kernel(xn, xe_attr, xe_src, xe_dst, fc1_w, fc1_b, dl_w1, dl_w2):
    npad = _E_PAD - _E
    src_i = xe_src.astype(jnp.int32)
    dst_i = xe_dst.astype(jnp.int32)
    gsrc = _chunk_idx(jnp.pad(src_i, (0, npad)))
    gdst = _chunk_idx(jnp.pad(dst_i, (0, npad)))
    ssrc = _local_idx(jnp.pad(src_i, (0, npad), constant_values=_N))
    sdst = _local_idx(jnp.pad(dst_i, (0, npad), constant_values=_N))

    xs, xd = _sc_gather(xn, gsrc, gdst)
    a_dst, a_src = _tc_edges(xe_attr, xs, xd, fc1_w.T, fc1_b[None, :],
                             dl_w1.T.astype(jnp.bfloat16))
    zrows = jnp.zeros((_ACC_R, _D), jnp.float32)
    acc = _sc_scatter(a_dst, a_src, sdst, ssrc, zrows)
    return acc[:_N]


# speedup vs baseline: 1.0323x; 1.0323x over previous
import functools

import jax
import jax.numpy as jnp
from jax import lax
from jax.experimental import pallas as pl
from jax.experimental.pallas import tpu as pltpu
from jax.experimental.pallas import tpu_sc as plsc

_N = 10000
_E = 320000
_D = 128
_A = 33

_NC = 2
_NS = 16
_NW = _NC * _NS
_E_PAD = 327680
_NCH = _E_PAD // 128
_HALF = 5120
_ACC_R = 5248
_T = 512

_NBUF = 5


def _sc_gather(xn, idx_all):
    mesh = plsc.VectorSubcoreMesh(core_axis_name="c", subcore_axis_name="s")
    tpw = 2 * _NCH // _NW
    half_t = tpw // 2

    @functools.partial(
        pl.kernel,
        out_type=jax.ShapeDtypeStruct((2 * _E_PAD, _D), jnp.float32),
        mesh=mesh,
        scratch_types=[pltpu.VMEM((tpw, 128), jnp.int32),
                       pltpu.VMEM((_NBUF * 128, _D), jnp.float32)]
        + [pltpu.SemaphoreType.DMA] * (2 * _NBUF),
    )
    def k(xn_hbm, idx_hbm, out_hbm, idx_all_v, rows_v,
          g0, g1, g2, g3, g4, o0, o1, o2, o3, o4):
        sg = [g0, g1, g2, g3, g4]
        so = [o0, o1, o2, o3, o4]
        w = lax.axis_index("s") * _NC + lax.axis_index("c")

        pltpu.sync_copy(idx_hbm.at[pl.ds(w * half_t, half_t)],
                        idx_all_v.at[pl.ds(0, half_t)])
        pltpu.sync_copy(idx_hbm.at[pl.ds(_NCH + w * half_t, half_t)],
                        idx_all_v.at[pl.ds(half_t, half_t)])

        def out_off(t):
            p = t // half_t
            return p * _E_PAD + w * (half_t * 128) + (t - p * half_t) * 128

        def fire_gather(t, r):
            pltpu.async_copy(xn_hbm.at[idx_all_v.at[t]],
                             rows_v.at[pl.ds(r * 128, 128)], sg[r])

        for r in range(2):
            fire_gather(r, r)

        def outer(q, carry):
            for r in range(_NBUF):
                t = q * _NBUF + r
                pltpu.make_async_copy(
                    xn_hbm.at[idx_all_v.at[t]],
                    rows_v.at[pl.ds(r * 128, 128)], sg[r]).wait()
                pltpu.async_copy(rows_v.at[pl.ds(r * 128, 128)],
                                 out_hbm.at[pl.ds(out_off(t), 128)], so[r])
                r2 = (r + 2) % _NBUF

                @pl.when(jnp.logical_and(t >= 3, t + 2 < tpw))
                def _():
                    pltpu.make_async_copy(
                        rows_v.at[pl.ds(r2 * 128, 128)],
                        out_hbm.at[pl.ds(0, 128)], so[r2]).wait()

                @pl.when(t + 2 < tpw)
                def _():
                    fire_gather(t + 2, r2)
            return carry

        lax.fori_loop(0, tpw // _NBUF, outer, 0)
        for r in range(_NBUF):
            pltpu.make_async_copy(rows_v.at[pl.ds(r * 128, 128)],
                                  out_hbm.at[pl.ds(0, 128)], so[r]).wait()

    return k(xn, idx_all)


def _sc_scatter(rows2, idx_all, zrows):
    mesh = plsc.VectorSubcoreMesh(core_axis_name="c", subcore_axis_name="s")
    z_per_sub = _ACC_R // _NS
    o_per_sub = _HALF // _NS
    cpw = _E_PAD // (_NS * _C_SCAT)
    epw = cpw * _C_SCAT

    @functools.partial(
        pl.kernel,
        out_type=jax.ShapeDtypeStruct((2 * _HALF, _D), jnp.float32),
        mesh=mesh,
        scratch_types=[pltpu.VMEM((8, 128), jnp.int32),
                       pltpu.VMEM((_C_SCAT, _D), jnp.float32),
                       pltpu.VMEM_SHARED((_ACC_R, _D), jnp.float32)],
    )
    def k(r_hbm, i_hbm, z_hbm, out_hbm, idx_v, rows_v, acc):
        c = lax.axis_index("c")
        s = lax.axis_index("s")
        base = s * epw

        pltpu.sync_copy(z_hbm.at[pl.ds(s * z_per_sub, z_per_sub)],
                        acc.at[pl.ds(s * z_per_sub, z_per_sub)])
        plsc.subcore_barrier()

        def run(arr):
            sec = (arr * _NC + c) * (_E_PAD // _C_SCAT)

            def body(ck, carry):
                off = arr * _E_PAD + base + ck * _C_SCAT
                pltpu.sync_copy(i_hbm.at[sec + s * cpw + ck], idx_v)
                pltpu.sync_copy(r_hbm.at[pl.ds(off, _C_SCAT)], rows_v)
                for j in range(_C_SCAT // 128):
                    pltpu.sync_copy(rows_v.at[pl.ds(j * 128, 128)],
                                    acc.at[idx_v.at[j]], add=True)
                return carry

            lax.fori_loop(0, cpw, body, 0)

        run(0)
        run(1)
        plsc.subcore_barrier()
        pltpu.sync_copy(acc.at[pl.ds(s * o_per_sub, o_per_sub)],
                        out_hbm.at[pl.ds(c * _HALF + s * o_per_sub,
                                         o_per_sub)])

    return k(rows2, idx_all, zrows)


_C_SCAT = 640


def _tc_edge_body(attr_ref, xs_ref, xd_ref, w1_ref, b1_ref, w2_ref,
                  out_ref):
    attr = attr_ref[...]
    W = jnp.dot(attr, w1_ref[...], preferred_element_type=jnp.float32)
    W = W + b1_ref[...]
    W = W * jax.nn.sigmoid(W)
    xs = xs_ref[...]
    xd = xd_ref[...]
    g = W * (xs - xd)
    a = W * (xs + xd) * 0.5
    dxe = jnp.concatenate([g, a, g * a, g * g, a * a], axis=1)
    w2 = w2_ref[...]
    x = jnp.tanh(dxe).astype(jnp.bfloat16)
    x = jnp.dot(x, w2, preferred_element_type=jnp.float32)
    x = x - jnp.mean(x, axis=1, keepdims=True)
    x = x * lax.rsqrt(jnp.sum(x * x, axis=1, keepdims=True) + 0.001)
    x = jnp.tanh(x).astype(jnp.bfloat16)
    x = jnp.dot(x, w2, preferred_element_type=jnp.float32)
    x = jnp.tanh(x)
    x0 = x[:, :_D]
    s = 0.5 * (x[:, _D:2 * _D] + x[:, 2 * _D:3 * _D]
               + x[:, 3 * _D:4 * _D] + x[:, 4 * _D:])
    out_ref[0] = W * (s + x0)
    out_ref[1] = W * (s - x0)


def _tc_edges(xe_attr, xsd, fc1_wT, fc1_b2, dl_w1T):
    grid = _E_PAD // _T
    last_real = _E // _T - 1
    nt = _E_PAD // _T
    return pl.pallas_call(
        _tc_edge_body,
        grid=(grid,),
        in_specs=[
            pl.BlockSpec((_T, _A), lambda i: (jnp.minimum(i, last_real), 0)),
            pl.BlockSpec((_T, _D), lambda i: (i, 0)),
            pl.BlockSpec((_T, _D), lambda i: (nt + i, 0)),
            pl.BlockSpec((_A, _D), lambda i: (0, 0)),
            pl.BlockSpec((1, _D), lambda i: (0, 0)),
            pl.BlockSpec((5 * _D, 5 * _D), lambda i: (0, 0)),
        ],
        out_specs=pl.BlockSpec((2, _T, _D), lambda i: (0, i, 0)),
        out_shape=jax.ShapeDtypeStruct((2, _E_PAD, _D), jnp.float32),
        compiler_params=pltpu.CompilerParams(
            dimension_semantics=("arbitrary",)),
    )(xe_attr, xsd, xsd, fc1_wT, fc1_b2, dl_w1T)


def _chunk_idx(idx_pad):
    a = idx_pad.reshape(-1, _C_SCAT // 128, 128)
    return jnp.pad(a, ((0, 0), (0, 8 - _C_SCAT // 128), (0, 0)))


def _local_idx(idx_pad):
    halves = []
    for core in range(_NC):
        loc = idx_pad - core * _HALF
        ok = (loc >= 0) & (loc < _HALF)
        halves.append(jnp.where(ok, loc, _HALF))
    return jnp.stack(halves)


def kernel(xn, xe_attr, xe_src, xe_dst, fc1_w, fc1_b, dl_w1, dl_w2):
    npad = _E_PAD - _E
    src_i = jnp.pad(xe_src.astype(jnp.int32), (0, npad), constant_values=_N)
    dst_i = jnp.pad(xe_dst.astype(jnp.int32), (0, npad), constant_values=_N)

    gidx = jnp.concatenate([jnp.where(src_i < _N, src_i, 0),
                            jnp.where(dst_i < _N, dst_i, 0)]
                           ).reshape(2 * _NCH, 128)
    sidx = _chunk_idx(jnp.concatenate(
        [_local_idx(dst_i), _local_idx(src_i)]).reshape(-1))

    xsd = _sc_gather(xn, gidx)
    out2 = _tc_edges(xe_attr, xsd, fc1_w.T,
                     fc1_b[None, :], dl_w1.T.astype(jnp.bfloat16))
    zrows = jnp.zeros((_ACC_R, _D), jnp.float32)
    acc = _sc_scatter(out2.reshape(2 * _E_PAD, _D), sidx, zrows)
    return acc[:_N]


# speedup vs baseline: 1.0890x; 1.0550x over previous
import functools

import jax
import jax.numpy as jnp
from jax import lax
from jax.experimental import pallas as pl
from jax.experimental.pallas import tpu as pltpu
from jax.experimental.pallas import tpu_sc as plsc

_N = 10000
_E = 320000
_D = 128
_A = 33

_NC = 2
_NS = 16
_E_PAD = 327680
_NCH = _E_PAD // 128
_HALF = 5120
_ACC_R = 5248
_T = 512

_NBUF = 5
_T_SC0 = 240
_T_SC1 = 80
_IDX_PAD = _NS * (_T_SC0 + _T_SC1) + _T_SC0


def _sc_gather(xn, idx_all):
    mesh = plsc.VectorSubcoreMesh(core_axis_name="c", subcore_axis_name="s")

    @functools.partial(
        pl.kernel,
        out_type=jax.ShapeDtypeStruct((2 * _E_PAD, _D), jnp.float32),
        mesh=mesh,
        scratch_types=[pltpu.VMEM((_T_SC0, 128), jnp.int32),
                       pltpu.VMEM((_NBUF * 128, _D), jnp.float32)]
        + [pltpu.SemaphoreType.DMA] * (2 * _NBUF),
    )
    def k(xn_hbm, idx_hbm, out_hbm, idx_all_v, rows_v,
          g0, g1, g2, g3, g4, o0, o1, o2, o3, o4):
        sg = [g0, g1, g2, g3, g4]
        so = [o0, o1, o2, o3, o4]
        c = lax.axis_index("c")
        s = lax.axis_index("s")
        cnt = jnp.where(c == 0, _T_SC0, _T_SC1)
        start = c * (_NS * _T_SC0) + s * cnt

        pltpu.sync_copy(idx_hbm.at[pl.ds(start, _T_SC0)], idx_all_v)

        def fire_gather(t, r):
            pltpu.async_copy(xn_hbm.at[idx_all_v.at[t]],
                             rows_v.at[pl.ds(r * 128, 128)], sg[r])

        for r in range(2):
            fire_gather(r, r)

        def outer(q, carry):
            for r in range(_NBUF):
                t = q * _NBUF + r
                pltpu.make_async_copy(
                    xn_hbm.at[idx_all_v.at[t]],
                    rows_v.at[pl.ds(r * 128, 128)], sg[r]).wait()
                pltpu.async_copy(
                    rows_v.at[pl.ds(r * 128, 128)],
                    out_hbm.at[pl.ds((start + t) * 128, 128)], so[r])
                r2 = (r + 2) % _NBUF

                @pl.when(jnp.logical_and(t >= 3, t + 2 < cnt))
                def _():
                    pltpu.make_async_copy(
                        rows_v.at[pl.ds(r2 * 128, 128)],
                        out_hbm.at[pl.ds(0, 128)], so[r2]).wait()

                @pl.when(t + 2 < cnt)
                def _():
                    fire_gather(t + 2, r2)
            return carry

        lax.fori_loop(0, cnt // _NBUF, outer, 0)
        for r in range(_NBUF):
            pltpu.make_async_copy(rows_v.at[pl.ds(r * 128, 128)],
                                  out_hbm.at[pl.ds(0, 128)], so[r]).wait()

    return k(xn, idx_all)


def _sc_scatter(rows2, idx_all, zrows):
    mesh = plsc.VectorSubcoreMesh(core_axis_name="c", subcore_axis_name="s")
    z_per_sub = _ACC_R // _NS
    o_per_sub = _HALF // _NS
    cps = _NCH // _NS
    tps = 2 * cps

    nbuf = 4

    @functools.partial(
        pl.kernel,
        out_type=jax.ShapeDtypeStruct((2 * _HALF, _D), jnp.float32),
        mesh=mesh,
        scratch_types=[pltpu.VMEM((cps, 128), jnp.int32),
                       pltpu.VMEM((nbuf * 128, _D), jnp.float32),
                       pltpu.VMEM_SHARED((_ACC_R, _D), jnp.float32)]
        + [pltpu.SemaphoreType.DMA] * nbuf,
    )
    def k(r_hbm, i_hbm, z_hbm, out_hbm, idx_all_v, rows_v, acc,
          l0, l1, l2, l3):
        sl = [l0, l1, l2, l3]
        c = lax.axis_index("c")
        s = lax.axis_index("s")

        pltpu.sync_copy(z_hbm.at[pl.ds(s * z_per_sub, z_per_sub)],
                        acc.at[pl.ds(s * z_per_sub, z_per_sub)])
        plsc.subcore_barrier()

        def phase(arr):
            pltpu.sync_copy(
                i_hbm.at[pl.ds((arr * _NC + c) * _NCH + s * cps, cps)],
                idx_all_v)
            base = arr * _E_PAD + s * (cps * 128)

            def fire_load(t, r):
                pltpu.async_copy(r_hbm.at[pl.ds(base + t * 128, 128)],
                                 rows_v.at[pl.ds(r * 128, 128)], sl[r])

            for r in range(2):
                fire_load(r, r)

            def outer(q, carry):
                for r in range(nbuf):
                    t = q * nbuf + r
                    pltpu.make_async_copy(
                        r_hbm.at[pl.ds(0, 128)],
                        rows_v.at[pl.ds(r * 128, 128)], sl[r]).wait()
                    r2 = (r + 2) % nbuf

                    @pl.when(t + 2 < cps)
                    def _():
                        fire_load(t + 2, r2)

                    pltpu.sync_copy(rows_v.at[pl.ds(r * 128, 128)],
                                    acc.at[idx_all_v.at[t]], add=True)
                return carry

            lax.fori_loop(0, cps // nbuf, outer, 0)

        phase(0)
        phase(1)
        plsc.subcore_barrier()
        pltpu.sync_copy(acc.at[pl.ds(s * o_per_sub, o_per_sub)],
                        out_hbm.at[pl.ds(c * _HALF + s * o_per_sub,
                                         o_per_sub)])

    return k(rows2, idx_all, zrows)


def _tc_edge_body(attr_ref, xs_ref, xd_ref, w1_ref, b1_ref, w2_ref,
                  out_ref):
    attr = attr_ref[...]
    W = jnp.dot(attr, w1_ref[...], preferred_element_type=jnp.float32)
    W = W + b1_ref[...]
    W = W * jax.nn.sigmoid(W)
    xs = xs_ref[...]
    xd = xd_ref[...]
    g = W * (xs - xd)
    a = W * (xs + xd) * 0.5
    dxe = jnp.concatenate([g, a, g * a, g * g, a * a], axis=1)
    w2 = w2_ref[...]
    x = jnp.tanh(dxe.astype(jnp.bfloat16))
    x = jnp.dot(x, w2, preferred_element_type=jnp.float32)
    x = x - jnp.mean(x, axis=1, keepdims=True)
    x = x * lax.rsqrt(jnp.sum(x * x, axis=1, keepdims=True) + 0.001)
    x = jnp.tanh(x.astype(jnp.bfloat16))
    x = jnp.dot(x, w2, preferred_element_type=jnp.float32)
    x = jnp.tanh(x)
    x0 = x[:, :_D]
    s = 0.5 * (x[:, _D:2 * _D] + x[:, 2 * _D:3 * _D]
               + x[:, 3 * _D:4 * _D] + x[:, 4 * _D:])
    out_ref[0] = W * (s + x0)
    out_ref[1] = W * (s - x0)


def _tc_edges(xe_attr, xsd, fc1_wT, fc1_b2, dl_w1T):
    grid = _E_PAD // _T
    last_real = _E // _T - 1
    nt = _E_PAD // _T
    return pl.pallas_call(
        _tc_edge_body,
        grid=(grid,),
        in_specs=[
            pl.BlockSpec((_T, _A), lambda i: (jnp.minimum(i, last_real), 0)),
            pl.BlockSpec((_T, _D), lambda i: (i, 0)),
            pl.BlockSpec((_T, _D), lambda i: (nt + i, 0)),
            pl.BlockSpec((_A, _D), lambda i: (0, 0)),
            pl.BlockSpec((1, _D), lambda i: (0, 0)),
            pl.BlockSpec((5 * _D, 5 * _D), lambda i: (0, 0)),
        ],
        out_specs=pl.BlockSpec((2, _T, _D), lambda i: (0, i, 0)),
        out_shape=jax.ShapeDtypeStruct((2, _E_PAD, _D), jnp.float32),
        compiler_params=pltpu.CompilerParams(
            dimension_semantics=("arbitrary",)),
    )(xe_attr, xsd, xsd, fc1_wT, fc1_b2, dl_w1T)


def _local_idx(idx_pad):
    halves = []
    for core in range(_NC):
        loc = idx_pad - core * _HALF
        ok = (loc >= 0) & (loc < _HALF)
        halves.append(jnp.where(ok, loc, _HALF))
    return jnp.stack(halves)


def kernel(xn, xe_attr, xe_src, xe_dst, fc1_w, fc1_b, dl_w1, dl_w2):
    npad = _E_PAD - _E
    src_i = jnp.pad(xe_src.astype(jnp.int32), (0, npad), constant_values=_N)
    dst_i = jnp.pad(xe_dst.astype(jnp.int32), (0, npad), constant_values=_N)

    gidx = jnp.concatenate([jnp.where(src_i < _N, src_i, 0),
                            jnp.where(dst_i < _N, dst_i, 0)]
                           ).reshape(2 * _NCH, 128)
    gidx = jnp.pad(gidx, ((0, _IDX_PAD - 2 * _NCH), (0, 0)))
    sidx = jnp.concatenate(
        [_local_idx(dst_i), _local_idx(src_i)]).reshape(4 * _NCH, 128)

    xsd = _sc_gather(xn, gidx)
    out2 = _tc_edges(xe_attr, xsd, fc1_w.T,
                     fc1_b[None, :], dl_w1.T.astype(jnp.bfloat16))
    zrows = jnp.zeros((_ACC_R, _D), jnp.float32)
    acc = _sc_scatter(out2.reshape(2 * _E_PAD, _D), sidx, zrows)
    return acc[:_N]


# speedup vs baseline: 1.1738x; 1.0778x over previous
import functools

import jax
import jax.numpy as jnp
from jax import lax
from jax.experimental import pallas as pl
from jax.experimental.pallas import tpu as pltpu
from jax.experimental.pallas import tpu_sc as plsc

_N = 10000
_E = 320000
_D = 128
_A = 33

_NC = 2
_NS = 16
_E_PAD = 327680
_NCH = _E_PAD // 128
_HALF = 5120
_ACC_R = 5248
_T = 1024

_NBUF = 5
_T_SC0 = 240
_T_SC1 = 80
_IDX_PAD = _NS * (_T_SC0 + _T_SC1) + _T_SC0


def _sc_gather(xn, idx_all):
    mesh = plsc.VectorSubcoreMesh(core_axis_name="c", subcore_axis_name="s")

    @functools.partial(
        pl.kernel,
        out_type=jax.ShapeDtypeStruct((2 * _E_PAD, _D), jnp.float32),
        mesh=mesh,
        scratch_types=[pltpu.VMEM((_T_SC0, 128), jnp.int32),
                       pltpu.VMEM((_NBUF * 128, _D), jnp.float32)]
        + [pltpu.SemaphoreType.DMA] * (2 * _NBUF),
    )
    def k(xn_hbm, idx_hbm, out_hbm, idx_all_v, rows_v,
          g0, g1, g2, g3, g4, o0, o1, o2, o3, o4):
        sg = [g0, g1, g2, g3, g4]
        so = [o0, o1, o2, o3, o4]
        c = lax.axis_index("c")
        s = lax.axis_index("s")
        cnt = jnp.where(c == 0, _T_SC0, _T_SC1)
        start = c * (_NS * _T_SC0) + s * cnt

        pltpu.sync_copy(idx_hbm.at[pl.ds(start, _T_SC0)], idx_all_v)

        def fire_gather(t, r):
            pltpu.async_copy(xn_hbm.at[idx_all_v.at[t]],
                             rows_v.at[pl.ds(r * 128, 128)], sg[r])

        for r in range(2):
            fire_gather(r, r)

        def outer(q, carry):
            for r in range(_NBUF):
                t = q * _NBUF + r
                pltpu.make_async_copy(
                    xn_hbm.at[idx_all_v.at[t]],
                    rows_v.at[pl.ds(r * 128, 128)], sg[r]).wait()
                pltpu.async_copy(
                    rows_v.at[pl.ds(r * 128, 128)],
                    out_hbm.at[pl.ds((start + t) * 128, 128)], so[r])
                r2 = (r + 2) % _NBUF

                @pl.when(jnp.logical_and(t >= 3, t + 2 < cnt))
                def _():
                    pltpu.make_async_copy(
                        rows_v.at[pl.ds(r2 * 128, 128)],
                        out_hbm.at[pl.ds(0, 128)], so[r2]).wait()

                @pl.when(t + 2 < cnt)
                def _():
                    fire_gather(t + 2, r2)
            return carry

        lax.fori_loop(0, cnt // _NBUF, outer, 0)
        for r in range(_NBUF):
            pltpu.make_async_copy(rows_v.at[pl.ds(r * 128, 128)],
                                  out_hbm.at[pl.ds(0, 128)], so[r]).wait()

    return k(xn, idx_all)


def _sc_scatter(rows2, idx_all, zrows):
    mesh = plsc.VectorSubcoreMesh(core_axis_name="c", subcore_axis_name="s")
    z_per_sub = _ACC_R // _NS
    o_per_sub = _HALF // _NS
    cps = _NCH // _NS
    tps = 2 * cps

    nbuf = 4

    @functools.partial(
        pl.kernel,
        out_type=jax.ShapeDtypeStruct((2 * _HALF, _D), jnp.float32),
        mesh=mesh,
        scratch_types=[pltpu.VMEM((cps, 128), jnp.int32),
                       pltpu.VMEM((nbuf * 128, _D), jnp.float32),
                       pltpu.VMEM_SHARED((_ACC_R, _D), jnp.float32)]
        + [pltpu.SemaphoreType.DMA] * nbuf,
    )
    def k(r_hbm, i_hbm, z_hbm, out_hbm, idx_all_v, rows_v, acc,
          l0, l1, l2, l3):
        sl = [l0, l1, l2, l3]
        c = lax.axis_index("c")
        s = lax.axis_index("s")

        pltpu.sync_copy(z_hbm.at[pl.ds(s * z_per_sub, z_per_sub)],
                        acc.at[pl.ds(s * z_per_sub, z_per_sub)])
        plsc.subcore_barrier()

        def phase(arr):
            pltpu.sync_copy(
                i_hbm.at[pl.ds((arr * _NC + c) * _NCH + s * cps, cps)],
                idx_all_v)
            base = arr * _E_PAD + s * (cps * 128)

            def fire_load(t, r):
                pltpu.async_copy(r_hbm.at[pl.ds(base + t * 128, 128)],
                                 rows_v.at[pl.ds(r * 128, 128)], sl[r])

            for r in range(2):
                fire_load(r, r)

            def outer(q, carry):
                for r in range(nbuf):
                    t = q * nbuf + r
                    pltpu.make_async_copy(
                        r_hbm.at[pl.ds(0, 128)],
                        rows_v.at[pl.ds(r * 128, 128)], sl[r]).wait()
                    r2 = (r + 2) % nbuf

                    @pl.when(t + 2 < cps)
                    def _():
                        fire_load(t + 2, r2)

                    pltpu.sync_copy(rows_v.at[pl.ds(r * 128, 128)],
                                    acc.at[idx_all_v.at[t]], add=True)
                return carry

            lax.fori_loop(0, cps // nbuf, outer, 0)

        phase(0)
        phase(1)
        plsc.subcore_barrier()
        pltpu.sync_copy(acc.at[pl.ds(s * o_per_sub, o_per_sub)],
                        out_hbm.at[pl.ds(c * _HALF + s * o_per_sub,
                                         o_per_sub)])

    return k(rows2, idx_all, zrows)


def _tc_edge_body(attr_ref, xs_ref, xd_ref, w1_ref, b1_ref, w2_ref,
                  out_ref):
    attr = attr_ref[...]
    W = jnp.dot(attr, w1_ref[...], preferred_element_type=jnp.float32)
    W = W + b1_ref[...]
    W = W * jax.nn.sigmoid(W)
    xs = xs_ref[...]
    xd = xd_ref[...]
    g = W * (xs - xd)
    a = W * (xs + xd) * 0.5
    dxe = jnp.concatenate([g, a, g * a, g * g, a * a], axis=1)
    w2 = w2_ref[...]
    x = jnp.tanh(dxe.astype(jnp.bfloat16))
    x = jnp.dot(x, w2, preferred_element_type=jnp.float32)
    x = x - jnp.mean(x, axis=1, keepdims=True)
    x = x * lax.rsqrt(jnp.sum(x * x, axis=1, keepdims=True) + 0.001)
    x = jnp.tanh(x.astype(jnp.bfloat16))
    x = jnp.dot(x, w2, preferred_element_type=jnp.float32)
    x = jnp.tanh(x)
    x0 = x[:, :_D]
    s = 0.5 * (x[:, _D:2 * _D] + x[:, 2 * _D:3 * _D]
               + x[:, 3 * _D:4 * _D] + x[:, 4 * _D:])
    out_ref[0] = W * (s + x0)
    out_ref[1] = W * (s - x0)


def _tc_edges(xe_attr, xsd, fc1_wT, fc1_b2, dl_w1T):
    grid = _E_PAD // _T
    last_real = _E // _T - 1
    nt = _E_PAD // _T
    return pl.pallas_call(
        _tc_edge_body,
        grid=(grid,),
        in_specs=[
            pl.BlockSpec((_T, _A), lambda i: (jnp.minimum(i, last_real), 0)),
            pl.BlockSpec((_T, _D), lambda i: (i, 0)),
            pl.BlockSpec((_T, _D), lambda i: (nt + i, 0)),
            pl.BlockSpec((_A, _D), lambda i: (0, 0)),
            pl.BlockSpec((1, _D), lambda i: (0, 0)),
            pl.BlockSpec((5 * _D, 5 * _D), lambda i: (0, 0)),
        ],
        out_specs=pl.BlockSpec((2, _T, _D), lambda i: (0, i, 0)),
        out_shape=jax.ShapeDtypeStruct((2, _E_PAD, _D), jnp.float32),
        compiler_params=pltpu.CompilerParams(
            dimension_semantics=("arbitrary",)),
    )(xe_attr, xsd, xsd, fc1_wT, fc1_b2, dl_w1T)


def _local_idx(idx_pad):
    halves = []
    for core in range(_NC):
        loc = idx_pad - core * _HALF
        ok = (loc >= 0) & (loc < _HALF)
        halves.append(jnp.where(ok, loc, _HALF))
    return jnp.stack(halves)


def kernel(xn, xe_attr, xe_src, xe_dst, fc1_w, fc1_b, dl_w1, dl_w2):
    npad = _E_PAD - _E
    src_i = jnp.pad(xe_src.astype(jnp.int32), (0, npad), constant_values=_N)
    dst_i = jnp.pad(xe_dst.astype(jnp.int32), (0, npad), constant_values=_N)

    gidx = jnp.concatenate([jnp.where(src_i < _N, src_i, 0),
                            jnp.where(dst_i < _N, dst_i, 0)]
                           ).reshape(2 * _NCH, 128)
    gidx = jnp.pad(gidx, ((0, _IDX_PAD - 2 * _NCH), (0, 0)))
    sidx = jnp.concatenate(
        [_local_idx(dst_i), _local_idx(src_i)]).reshape(4 * _NCH, 128)

    xsd = _sc_gather(xn, gidx)
    out2 = _tc_edges(xe_attr, xsd, fc1_w.T,
                     fc1_b[None, :], dl_w1.T.astype(jnp.bfloat16))
    zrows = jnp.zeros((_ACC_R, _D), jnp.float32)
    acc = _sc_scatter(out2.reshape(2 * _E_PAD, _D), sidx, zrows)
    return acc[:_N]


# speedup vs baseline: 1.1747x; 1.0008x over previous
import functools

import jax
import jax.numpy as jnp
from jax import lax
from jax.experimental import pallas as pl
from jax.experimental.pallas import tpu as pltpu
from jax.experimental.pallas import tpu_sc as plsc

_N = 10000
_E = 320000
_D = 128
_A = 33

_NC = 2
_NS = 16
_E_PAD = 327680
_NCH = _E_PAD // 128
_HALF = 5120
_ACC_R = 5248
_T = 1024

_NBUF = 5
_T_SC0 = 240
_T_SC1 = 80
_IDX_PAD = _NS * (_T_SC0 + _T_SC1) + _T_SC0


def _sc_gather(xn, idx_all):
    mesh = plsc.VectorSubcoreMesh(core_axis_name="c", subcore_axis_name="s")

    @functools.partial(
        pl.kernel,
        out_type=jax.ShapeDtypeStruct((2 * _E_PAD, _D), jnp.float32),
        mesh=mesh,
        scratch_types=[pltpu.VMEM((_T_SC0, 128), jnp.int32),
                       pltpu.VMEM((_NBUF * 128, _D), jnp.float32)]
        + [pltpu.SemaphoreType.DMA] * (2 * _NBUF),
    )
    def k(xn_hbm, idx_hbm, out_hbm, idx_all_v, rows_v,
          g0, g1, g2, g3, g4, o0, o1, o2, o3, o4):
        sg = [g0, g1, g2, g3, g4]
        so = [o0, o1, o2, o3, o4]
        c = lax.axis_index("c")
        s = lax.axis_index("s")
        cnt = jnp.where(c == 0, _T_SC0, _T_SC1)
        start = c * (_NS * _T_SC0) + s * cnt

        pltpu.sync_copy(idx_hbm.at[pl.ds(start, _T_SC0)], idx_all_v)

        def fire_gather(t, r):
            pltpu.async_copy(xn_hbm.at[idx_all_v.at[t]],
                             rows_v.at[pl.ds(r * 128, 128)], sg[r])

        for r in range(2):
            fire_gather(r, r)

        def outer(q, carry):
            for r in range(_NBUF):
                t = q * _NBUF + r
                pltpu.make_async_copy(
                    xn_hbm.at[idx_all_v.at[t]],
                    rows_v.at[pl.ds(r * 128, 128)], sg[r]).wait()
                pltpu.async_copy(
                    rows_v.at[pl.ds(r * 128, 128)],
                    out_hbm.at[pl.ds((start + t) * 128, 128)], so[r])
                r2 = (r + 2) % _NBUF

                @pl.when(jnp.logical_and(t >= 3, t + 2 < cnt))
                def _():
                    pltpu.make_async_copy(
                        rows_v.at[pl.ds(r2 * 128, 128)],
                        out_hbm.at[pl.ds(0, 128)], so[r2]).wait()

                @pl.when(t + 2 < cnt)
                def _():
                    fire_gather(t + 2, r2)
            return carry

        lax.fori_loop(0, cnt // _NBUF, outer, 0)
        for r in range(_NBUF):
            pltpu.make_async_copy(rows_v.at[pl.ds(r * 128, 128)],
                                  out_hbm.at[pl.ds(0, 128)], so[r]).wait()

    return k(xn, idx_all)


def _sc_scatter(rows2, idx_all, zrows):
    mesh = plsc.VectorSubcoreMesh(core_axis_name="c", subcore_axis_name="s")
    z_per_sub = _ACC_R // _NS
    o_per_sub = _HALF // _NS
    cps = _NCH // _NS
    tps = 2 * cps

    nbuf = 4

    @functools.partial(
        pl.kernel,
        out_type=jax.ShapeDtypeStruct((2 * _HALF, _D), jnp.float32),
        mesh=mesh,
        scratch_types=[pltpu.VMEM((cps, 128), jnp.int32),
                       pltpu.VMEM((nbuf * 128, _D), jnp.float32),
                       pltpu.VMEM_SHARED((_ACC_R, _D), jnp.float32)]
        + [pltpu.SemaphoreType.DMA] * nbuf,
    )
    def k(r_hbm, i_hbm, z_hbm, out_hbm, idx_all_v, rows_v, acc,
          l0, l1, l2, l3):
        sl = [l0, l1, l2, l3]
        c = lax.axis_index("c")
        s = lax.axis_index("s")

        pltpu.sync_copy(z_hbm.at[pl.ds(s * z_per_sub, z_per_sub)],
                        acc.at[pl.ds(s * z_per_sub, z_per_sub)])
        plsc.subcore_barrier()

        def phase(arr):
            pltpu.sync_copy(
                i_hbm.at[pl.ds((arr * _NC + c) * _NCH + s * cps, cps)],
                idx_all_v)
            base = arr * _E_PAD + s * (cps * 128)

            def fire_load(t, r):
                pltpu.async_copy(r_hbm.at[pl.ds(base + t * 128, 128)],
                                 rows_v.at[pl.ds(r * 128, 128)], sl[r])

            for r in range(2):
                fire_load(r, r)

            def outer(q, carry):
                for r in range(nbuf):
                    t = q * nbuf + r
                    pltpu.make_async_copy(
                        r_hbm.at[pl.ds(0, 128)],
                        rows_v.at[pl.ds(r * 128, 128)], sl[r]).wait()
                    r2 = (r + 2) % nbuf

                    @pl.when(t + 2 < cps)
                    def _():
                        fire_load(t + 2, r2)

                    pltpu.sync_copy(rows_v.at[pl.ds(r * 128, 128)],
                                    acc.at[idx_all_v.at[t]], add=True)
                return carry

            lax.fori_loop(0, cps // nbuf, outer, 0)

        phase(0)
        phase(1)
        plsc.subcore_barrier()
        pltpu.sync_copy(acc.at[pl.ds(s * o_per_sub, o_per_sub)],
                        out_hbm.at[pl.ds(c * _HALF + s * o_per_sub,
                                         o_per_sub)])

    return k(rows2, idx_all, zrows)


def _tc_edge_body(attr_ref, xs_ref, xd_ref, w1_ref, b1_ref, w2_ref,
                  out_ref):
    attr = attr_ref[...]
    W = jnp.dot(attr, w1_ref[...], preferred_element_type=jnp.float32)
    W = W + b1_ref[...]
    W = W * jax.nn.sigmoid(W)
    xs = xs_ref[...]
    xd = xd_ref[...]
    g = W * (xs - xd)
    a = W * (xs + xd) * 0.5
    dxe = jnp.concatenate([g, a, g * a, g * g, a * a], axis=1)
    w2 = w2_ref[...]
    x = jnp.tanh(dxe.astype(jnp.bfloat16))
    x = jnp.dot(x, w2, preferred_element_type=jnp.float32)
    x = x - jnp.mean(x, axis=1, keepdims=True)
    x = x * lax.rsqrt(jnp.sum(x * x, axis=1, keepdims=True) + 0.001)
    x = jnp.tanh(x.astype(jnp.bfloat16))
    x = jnp.dot(x, w2, preferred_element_type=jnp.float32)
    x = jnp.tanh(x)
    x0 = x[:, :_D]
    s = 0.5 * (x[:, _D:2 * _D] + x[:, 2 * _D:3 * _D]
               + x[:, 3 * _D:4 * _D] + x[:, 4 * _D:])
    out_ref[0] = W * (s + x0)
    out_ref[1] = W * (s - x0)


def _tc_edges(xe_attr, xsd, fc1_wT, fc1_b2, dl_w1T):
    grid = _E_PAD // _T
    last_real = (_E + _T - 1) // _T - 1
    nt = _E_PAD // _T
    return pl.pallas_call(
        _tc_edge_body,
        grid=(grid,),
        in_specs=[
            pl.BlockSpec((_T, _A), lambda i: (jnp.minimum(i, last_real), 0)),
            pl.BlockSpec((_T, _D), lambda i: (i, 0)),
            pl.BlockSpec((_T, _D), lambda i: (nt + i, 0)),
            pl.BlockSpec((_A, _D), lambda i: (0, 0)),
            pl.BlockSpec((1, _D), lambda i: (0, 0)),
            pl.BlockSpec((5 * _D, 5 * _D), lambda i: (0, 0)),
        ],
        out_specs=pl.BlockSpec((2, _T, _D), lambda i: (0, i, 0)),
        out_shape=jax.ShapeDtypeStruct((2, _E_PAD, _D), jnp.float32),
        compiler_params=pltpu.CompilerParams(
            dimension_semantics=("arbitrary",)),
    )(xe_attr, xsd, xsd, fc1_wT, fc1_b2, dl_w1T)


def _local_idx(idx_pad):
    halves = []
    for core in range(_NC):
        loc = idx_pad - core * _HALF
        ok = (loc >= 0) & (loc < _HALF)
        halves.append(jnp.where(ok, loc, _HALF))
    return jnp.stack(halves)


def kernel(xn, xe_attr, xe_src, xe_dst, fc1_w, fc1_b, dl_w1, dl_w2):
    npad = _E_PAD - _E
    src_i = jnp.pad(xe_src.astype(jnp.int32), (0, npad), constant_values=_N)
    dst_i = jnp.pad(xe_dst.astype(jnp.int32), (0, npad), constant_values=_N)

    gidx = jnp.concatenate([jnp.where(src_i < _N, src_i, 0),
                            jnp.where(dst_i < _N, dst_i, 0)]
                           ).reshape(2 * _NCH, 128)
    gidx = jnp.pad(gidx, ((0, _IDX_PAD - 2 * _NCH), (0, 0)))
    sidx = jnp.concatenate(
        [_local_idx(dst_i), _local_idx(src_i)]).reshape(4 * _NCH, 128)

    xsd = _sc_gather(xn, gidx)
    out2 = _tc_edges(xe_attr, xsd, fc1_w.T,
                     fc1_b[None, :], dl_w1.T.astype(jnp.bfloat16))
    zrows = jnp.zeros((_ACC_R, _D), jnp.float32)
    acc = _sc_scatter(out2.reshape(2 * _E_PAD, _D), sidx, zrows)
    return acc[:_N]


# speedup vs baseline: 1.1749x; 1.0002x over previous
import functools

import jax
import jax.numpy as jnp
from jax import lax
from jax.experimental import pallas as pl
from jax.experimental.pallas import tpu as pltpu
from jax.experimental.pallas import tpu_sc as plsc

_N = 10000
_E = 320000
_D = 128
_A = 33

_NC = 2
_NS = 16
_E_PAD = 327680
_NCH = _E_PAD // 128
_HALF = 5120
_ACC_R = 5248
_T = 1024

_NBUF = 4
_T_SC0 = 240
_T_SC1 = 80
_IDX_PAD = _NS * (_T_SC0 + _T_SC1) + _T_SC0


_XNR = 10240
_DW = _D // 2


def _sc_gather(xn, idx_all):
    mesh = plsc.VectorSubcoreMesh(core_axis_name="c", subcore_axis_name="s")

    @functools.partial(
        pl.kernel,
        out_type=jax.ShapeDtypeStruct((2 * _E_PAD, _D), jnp.float32),
        mesh=mesh,
        scratch_types=[pltpu.VMEM((_T_SC0, 128), jnp.int32),
                       pltpu.VMEM((5 * 128, _D), jnp.float32)]
        + [pltpu.SemaphoreType.DMA] * 10,
    )
    def k(xn_hbm, idx_hbm, out_hbm, idx_all_v, rows_v,
          g0, g1, g2, g3, g4, o0, o1, o2, o3, o4):
        sg = [g0, g1, g2, g3, g4]
        so = [o0, o1, o2, o3, o4]
        c = lax.axis_index("c")
        s = lax.axis_index("s")
        cnt = jnp.where(c == 0, _T_SC0, _T_SC1)
        start = c * (_NS * _T_SC0) + s * cnt

        pltpu.sync_copy(idx_hbm.at[pl.ds(start, _T_SC0)], idx_all_v)

        def fire_gather(t, r):
            pltpu.async_copy(xn_hbm.at[idx_all_v.at[t]],
                             rows_v.at[pl.ds(r * 128, 128)], sg[r])

        for r in range(2):
            fire_gather(r, r)

        def outer(q, carry):
            for r in range(5):
                t = q * 5 + r
                pltpu.make_async_copy(
                    xn_hbm.at[idx_all_v.at[t]],
                    rows_v.at[pl.ds(r * 128, 128)], sg[r]).wait()
                pltpu.async_copy(
                    rows_v.at[pl.ds(r * 128, 128)],
                    out_hbm.at[pl.ds((start + t) * 128, 128)], so[r])
                r2 = (r + 2) % 5

                @pl.when(jnp.logical_and(t >= 3, t + 2 < cnt))
                def _():
                    pltpu.make_async_copy(
                        rows_v.at[pl.ds(r2 * 128, 128)],
                        out_hbm.at[pl.ds(0, 128)], so[r2]).wait()

                @pl.when(t + 2 < cnt)
                def _():
                    fire_gather(t + 2, r2)
            return carry

        lax.fori_loop(0, cnt // 5, outer, 0)
        for r in range(5):
            pltpu.make_async_copy(rows_v.at[pl.ds(r * 128, 128)],
                                  out_hbm.at[pl.ds(0, 128)], so[r]).wait()

    return k(xn, idx_all)


def _sc_scatter(rows2, idx_all, zrows):
    mesh = plsc.VectorSubcoreMesh(core_axis_name="c", subcore_axis_name="s")
    z_per_sub = _ACC_R // _NS
    o_per_sub = _HALF // _NS
    cps = _NCH // _NS
    tps = 2 * cps

    nbuf = 4

    @functools.partial(
        pl.kernel,
        out_type=jax.ShapeDtypeStruct((2 * _HALF, _D), jnp.float32),
        mesh=mesh,
        scratch_types=[pltpu.VMEM((cps, 128), jnp.int32),
                       pltpu.VMEM((nbuf * 128, _D), jnp.float32),
                       pltpu.VMEM_SHARED((_ACC_R, _D), jnp.float32)]
        + [pltpu.SemaphoreType.DMA] * nbuf,
    )
    def k(r_hbm, i_hbm, z_hbm, out_hbm, idx_all_v, rows_v, acc,
          l0, l1, l2, l3):
        sl = [l0, l1, l2, l3]
        c = lax.axis_index("c")
        s = lax.axis_index("s")

        pltpu.sync_copy(z_hbm.at[pl.ds(s * z_per_sub, z_per_sub)],
                        acc.at[pl.ds(s * z_per_sub, z_per_sub)])
        plsc.subcore_barrier()

        def phase(arr):
            pltpu.sync_copy(
                i_hbm.at[pl.ds((arr * _NC + c) * _NCH + s * cps, cps)],
                idx_all_v)
            base = arr * _E_PAD + s * (cps * 128)

            def fire_load(t, r):
                pltpu.async_copy(r_hbm.at[pl.ds(base + t * 128, 128)],
                                 rows_v.at[pl.ds(r * 128, 128)], sl[r])

            for r in range(2):
                fire_load(r, r)

            def outer(q, carry):
                for r in range(nbuf):
                    t = q * nbuf + r
                    pltpu.make_async_copy(
                        r_hbm.at[pl.ds(0, 128)],
                        rows_v.at[pl.ds(r * 128, 128)], sl[r]).wait()
                    r2 = (r + 2) % nbuf

                    @pl.when(t + 2 < cps)
                    def _():
                        fire_load(t + 2, r2)

                    pltpu.sync_copy(rows_v.at[pl.ds(r * 128, 128)],
                                    acc.at[idx_all_v.at[t]], add=True)
                return carry

            lax.fori_loop(0, cps // nbuf, outer, 0)

        phase(0)
        phase(1)
        plsc.subcore_barrier()
        pltpu.sync_copy(acc.at[pl.ds(s * o_per_sub, o_per_sub)],
                        out_hbm.at[pl.ds(c * _HALF + s * o_per_sub,
                                         o_per_sub)])

    return k(rows2, idx_all, zrows)


def _tc_edge_body(attr_ref, xs_ref, xd_ref, w1_ref, b1_ref, w2_ref,
                  out_ref):
    attr = attr_ref[...]
    W = jnp.dot(attr, w1_ref[...], preferred_element_type=jnp.float32)
    W = W + b1_ref[...]
    W = W * jax.nn.sigmoid(W)
    xs = xs_ref[...].astype(jnp.float32)
    xd = xd_ref[...].astype(jnp.float32)
    g = W * (xs - xd)
    a = W * (xs + xd) * 0.5
    dxe = jnp.concatenate([g, a, g * a, g * g, a * a], axis=1)
    w2 = w2_ref[...]
    x = jnp.tanh(dxe.astype(jnp.bfloat16))
    x = jnp.dot(x, w2, preferred_element_type=jnp.float32)
    x = x - jnp.mean(x, axis=1, keepdims=True)
    x = x * lax.rsqrt(jnp.sum(x * x, axis=1, keepdims=True) + 0.001)
    x = jnp.tanh(x.astype(jnp.bfloat16))
    x = jnp.dot(x, w2, preferred_element_type=jnp.float32)
    x = jnp.tanh(x)
    x0 = x[:, :_D]
    s = 0.5 * (x[:, _D:2 * _D] + x[:, 2 * _D:3 * _D]
               + x[:, 3 * _D:4 * _D] + x[:, 4 * _D:])
    out_ref[0] = W * (s + x0)
    out_ref[1] = W * (s - x0)


def _tc_edges(xe_attr, xsd, fc1_wT, fc1_b2, dl_w1T):
    grid = _E_PAD // _T
    last_real = (_E + _T - 1) // _T - 1
    nt = _E_PAD // _T
    return pl.pallas_call(
        _tc_edge_body,
        grid=(grid,),
        in_specs=[
            pl.BlockSpec((_T, _A), lambda i: (jnp.minimum(i, last_real), 0)),
            pl.BlockSpec((_T, _D), lambda i: (i, 0)),
            pl.BlockSpec((_T, _D), lambda i: (nt + i, 0)),
            pl.BlockSpec((_A, _D), lambda i: (0, 0)),
            pl.BlockSpec((1, _D), lambda i: (0, 0)),
            pl.BlockSpec((5 * _D, 5 * _D), lambda i: (0, 0)),
        ],
        out_specs=pl.BlockSpec((2, _T, _D), lambda i: (0, i, 0)),
        out_shape=jax.ShapeDtypeStruct((2, _E_PAD, _D), jnp.float32),
        compiler_params=pltpu.CompilerParams(
            dimension_semantics=("arbitrary",)),
    )(xe_attr, xsd, xsd, fc1_wT, fc1_b2, dl_w1T)


def _local_idx(idx_pad):
    halves = []
    for core in range(_NC):
        loc = idx_pad - core * _HALF
        ok = (loc >= 0) & (loc < _HALF)
        halves.append(jnp.where(ok, loc, _HALF))
    return jnp.stack(halves)


def kernel(xn, xe_attr, xe_src, xe_dst, fc1_w, fc1_b, dl_w1, dl_w2):
    npad = _E_PAD - _E
    src_i = jnp.pad(xe_src.astype(jnp.int32), (0, npad), constant_values=_N)
    dst_i = jnp.pad(xe_dst.astype(jnp.int32), (0, npad), constant_values=_N)

    gidx = jnp.concatenate([jnp.where(src_i < _N, src_i, 0),
                            jnp.where(dst_i < _N, dst_i, 0)]
                           ).reshape(2 * _NCH, 128)
    gidx = jnp.pad(gidx, ((0, _IDX_PAD - 2 * _NCH), (0, 0)))
    sidx = jnp.concatenate(
        [_local_idx(dst_i), _local_idx(src_i)]).reshape(4 * _NCH, 128)

    xsd = _sc_gather(xn, gidx)
    out2 = _tc_edges(xe_attr, xsd, fc1_w.T,
                     fc1_b[None, :], dl_w1.T.astype(jnp.bfloat16))
    zrows = jnp.zeros((_ACC_R, _D), jnp.float32)
    acc = _sc_scatter(out2.reshape(2 * _E_PAD, _D), sidx, zrows)
    return acc[:_N]
